# baseline (speedup 1.0000x reference)
import functools

import jax
import jax.numpy as jnp
from jax.experimental import pallas as pl
from jax.experimental.pallas import tpu as pltpu

Q = 1024
D = 64
KB = 8192
RC = 1024
LW = 128


def _nn_kernel(q_ref, k_ref, out_ref, minval_ref, minidx_ref, mineven_ref,
               qm_ref, qsq_ref, *, nsteps, n_keys):
    step = pl.program_id(0)

    @pl.when(step == 0)
    def _init():
        minval_ref[...] = jnp.full(minval_ref.shape, jnp.inf, jnp.float32)
        minidx_ref[...] = jnp.zeros(minidx_ref.shape, jnp.float32)
        mineven_ref[...] = jnp.zeros(mineven_ref.shape, jnp.float32)
        qa = q_ref[...]
        qm_ref[...] = qa * (-2.0)
        qsq_ref[...] = jnp.sum(qa * qa, axis=1, keepdims=True)

    rowk = jax.lax.broadcasted_iota(jnp.int32, (KB, 1), 0)
    k = jnp.where((step * KB + rowk) < n_keys, k_ref[...], 0.0)
    k_sq = jnp.sum(k * k, axis=1)[None, :]
    lane1 = jax.lax.broadcasted_iota(jnp.int32, (1, KB), 1)
    valid = (step * KB + lane1) < n_keys
    k_sq = jnp.where(valid, k_sq, jnp.inf)

    ncols = KB // LW
    base0 = (step * KB).astype(jnp.float32)

    for rc in range(Q // RC):
        rows = pl.ds(rc * RC, RC)
        q_sq = qsq_ref[rows, :]
        qm = qm_ref[rows, :]
        qk2 = jax.lax.dot_general(qm, k, (((1,), (1,)), ((), ())),
                                  preferred_element_type=jnp.float32)
        m = minval_ref[rows, :]
        idx = minidx_ref[rows, :]
        weven = mineven_ref[rows, :]
        for c in range(0, ncols, 2):
            ta = jax.lax.slice(qk2, (0, c * LW), (RC, (c + 1) * LW))
            tb = jax.lax.slice(qk2, (0, (c + 1) * LW), (RC, (c + 2) * LW))
            ka = jax.lax.slice(k_sq, (0, c * LW), (1, (c + 1) * LW))
            kb = jax.lax.slice(k_sq, (0, (c + 1) * LW), (1, (c + 2) * LW))
            d2a = (q_sq + ta) + ka
            d2b = (q_sq + tb) + kb
            mp = jnp.minimum(d2a, d2b)
            upd = mp < m
            m = jnp.where(upd, mp, m)
            idx = jnp.where(upd, base0 + jnp.float32(c * LW), idx)
            weven = jnp.where(upd, d2a, weven)
        minval_ref[rows, :] = m
        minidx_ref[rows, :] = idx
        mineven_ref[rows, :] = weven

    @pl.when(step == nsteps - 1)
    def _done():
        lane_f = jax.lax.broadcasted_iota(jnp.int32, (Q, LW), 1).astype(jnp.float32)
        m = minval_ref[...]
        idx = minidx_ref[...]
        weven = mineven_ref[...]
        off = jnp.where(weven == m, jnp.float32(0.0), jnp.float32(LW))
        gidx = (idx + off) + lane_f
        bmin = jnp.min(m, axis=1, keepdims=True)
        cand = jnp.where(m == bmin, gidx, jnp.float32(3e7))
        out_ref[...] = jnp.min(cand, axis=1, keepdims=True).astype(jnp.int32)


def kernel(queries, keys):
    n_keys = keys.shape[0]
    nsteps = (n_keys + KB - 1) // KB
    out = pl.pallas_call(
        functools.partial(_nn_kernel, nsteps=nsteps, n_keys=n_keys),
        grid=(nsteps,),
        in_specs=[
            pl.BlockSpec((Q, D), lambda i: (0, 0)),
            pl.BlockSpec((KB, D), lambda i: (i, 0)),
        ],
        out_specs=pl.BlockSpec((Q, 1), lambda i: (0, 0)),
        out_shape=jax.ShapeDtypeStruct((Q, 1), jnp.int32),
        scratch_shapes=[
            pltpu.VMEM((Q, LW), jnp.float32),
            pltpu.VMEM((Q, LW), jnp.float32),
            pltpu.VMEM((Q, LW), jnp.float32),
            pltpu.VMEM((Q, D), jnp.float32),
            pltpu.VMEM((Q, 1), jnp.float32),
        ],
    )(queries, keys)
    return out.reshape(Q)


# speedup vs baseline: 1.0167x; 1.0167x over previous
import functools

import jax
import jax.numpy as jnp
from jax.experimental import pallas as pl
from jax.experimental.pallas import tpu as pltpu

Q = 1024
D = 64
KB = 4096
RC = 1024
LW = 128


def _nn_kernel(q_ref, k_ref, out_ref, minval_ref, minidx_ref, mineven_ref,
               qm_ref, qsq_ref, *, nsteps, n_keys):
    step = pl.program_id(0)

    @pl.when(step == 0)
    def _init():
        minval_ref[...] = jnp.full(minval_ref.shape, jnp.inf, jnp.float32)
        minidx_ref[...] = jnp.zeros(minidx_ref.shape, jnp.float32)
        mineven_ref[...] = jnp.zeros(mineven_ref.shape, jnp.float32)
        qa = q_ref[...]
        qm_ref[...] = qa * (-2.0)
        qsq_ref[...] = jnp.sum(qa * qa, axis=1, keepdims=True)

    rowk = jax.lax.broadcasted_iota(jnp.int32, (KB, 1), 0)
    k = jnp.where((step * KB + rowk) < n_keys, k_ref[...], 0.0)
    k_sq = jnp.sum(k * k, axis=1)[None, :]
    lane1 = jax.lax.broadcasted_iota(jnp.int32, (1, KB), 1)
    valid = (step * KB + lane1) < n_keys
    k_sq = jnp.where(valid, k_sq, jnp.inf)

    ncols = KB // LW
    base0 = (step * KB).astype(jnp.float32)

    for rc in range(Q // RC):
        rows = pl.ds(rc * RC, RC)
        q_sq = qsq_ref[rows, :]
        qm = qm_ref[rows, :]
        qk2 = jax.lax.dot_general(qm, k, (((1,), (1,)), ((), ())),
                                  preferred_element_type=jnp.float32)
        m = minval_ref[rows, :]
        idx = minidx_ref[rows, :]
        weven = mineven_ref[rows, :]
        for c in range(0, ncols, 2):
            ta = jax.lax.slice(qk2, (0, c * LW), (RC, (c + 1) * LW))
            tb = jax.lax.slice(qk2, (0, (c + 1) * LW), (RC, (c + 2) * LW))
            ka = jax.lax.slice(k_sq, (0, c * LW), (1, (c + 1) * LW))
            kb = jax.lax.slice(k_sq, (0, (c + 1) * LW), (1, (c + 2) * LW))
            d2a = (q_sq + ta) + ka
            d2b = (q_sq + tb) + kb
            mp = jnp.minimum(d2a, d2b)
            upd = mp < m
            m = jnp.where(upd, mp, m)
            idx = jnp.where(upd, base0 + jnp.float32(c * LW), idx)
            weven = jnp.where(upd, d2a, weven)
        minval_ref[rows, :] = m
        minidx_ref[rows, :] = idx
        mineven_ref[rows, :] = weven

    @pl.when(step == nsteps - 1)
    def _done():
        lane_f = jax.lax.broadcasted_iota(jnp.int32, (Q, LW), 1).astype(jnp.float32)
        m = minval_ref[...]
        idx = minidx_ref[...]
        weven = mineven_ref[...]
        off = jnp.where(weven == m, jnp.float32(0.0), jnp.float32(LW))
        gidx = (idx + off) + lane_f
        bmin = jnp.min(m, axis=1, keepdims=True)
        cand = jnp.where(m == bmin, gidx, jnp.float32(3e7))
        out_ref[...] = jnp.min(cand, axis=1, keepdims=True).astype(jnp.int32)


def kernel(queries, keys):
    n_keys = keys.shape[0]
    nsteps = (n_keys + KB - 1) // KB
    out = pl.pallas_call(
        functools.partial(_nn_kernel, nsteps=nsteps, n_keys=n_keys),
        grid=(nsteps,),
        in_specs=[
            pl.BlockSpec((Q, D), lambda i: (0, 0)),
            pl.BlockSpec((KB, D), lambda i: (i, 0)),
        ],
        out_specs=pl.BlockSpec((Q, 1), lambda i: (0, 0)),
        out_shape=jax.ShapeDtypeStruct((Q, 1), jnp.int32),
        scratch_shapes=[
            pltpu.VMEM((Q, LW), jnp.float32),
            pltpu.VMEM((Q, LW), jnp.float32),
            pltpu.VMEM((Q, LW), jnp.float32),
            pltpu.VMEM((Q, D), jnp.float32),
            pltpu.VMEM((Q, 1), jnp.float32),
        ],
    )(queries, keys)
    return out.reshape(Q)
